# baseline (speedup 1.0000x reference)
import jax
import jax.numpy as jnp
from jax import lax
from jax.experimental import pallas as pl
from jax.experimental.pallas import tpu as pltpu
from jax.experimental.pallas import tpu_sc as plsc

EMBED_DIM = 64
PAD_DIM = 128
SCALE = 8.0

_info = plsc.get_sparse_core_info()
NC, NS, L = _info.num_cores, _info.num_subcores, _info.num_lanes
NW = NC * NS
BLK = 128
SEQ = 200


def _emb_body(table_hbm, idxT_hbm, out_hbm,
              idx_all, g0buf, g1buf, o0buf, o1buf, sem0, sem1):
    wid = lax.axis_index("s") * NC + lax.axis_index("c")
    b0 = wid * BLK

    pltpu.sync_copy(idxT_hbm.at[:, pl.ds(b0, BLK)], idx_all)

    def fire(s, gbuf, sem):
        pltpu.async_copy(table_hbm.at[idx_all.at[s]], gbuf, sem)

    def drain(gbuf, sem):
        pltpu.make_async_copy(table_hbm.at[pl.ds(0, BLK)], gbuf, sem).wait()

    def transpose_scale(gbuf, obuf):
        @plsc.parallel_loop(0, EMBED_DIM, unroll=4)
        def _(d):
            for j in range(BLK // L):
                rows = jax.lax.iota(jnp.int32, L) + j * L
                cols = jnp.full((L,), 0, jnp.int32) + d
                v = plsc.load_gather(gbuf, [rows, cols])
                obuf[d, pl.ds(j * L, L)] = v * SCALE

    def store(s, obuf):
        pltpu.sync_copy(obuf, out_hbm.at[s, :, pl.ds(b0, BLK)])

    fire(0, g0buf, sem0)

    def pair(p, carry):
        s0 = p * 2
        fire(s0 + 1, g1buf, sem1)
        drain(g0buf, sem0)
        transpose_scale(g0buf, o0buf)
        store(s0, o0buf)
        fire(s0 + 2, g0buf, sem0)
        drain(g1buf, sem1)
        transpose_scale(g1buf, o1buf)
        store(s0 + 1, o1buf)
        return carry

    lax.fori_loop(0, SEQ // 2 - 1, pair, 0)

    s0 = SEQ - 2
    fire(s0 + 1, g1buf, sem1)
    drain(g0buf, sem0)
    transpose_scale(g0buf, o0buf)
    store(s0, o0buf)
    drain(g1buf, sem1)
    transpose_scale(g1buf, o1buf)
    store(s0 + 1, o1buf)


def kernel(tokens, table):
    n_b, n_s = tokens.shape
    assert n_b == NW * BLK and n_s == SEQ
    idxT = tokens.T
    vocab = table.shape[0]
    table128 = jnp.concatenate(
        [table, jnp.zeros((vocab, PAD_DIM - EMBED_DIM), table.dtype)], axis=1)

    mesh = plsc.VectorSubcoreMesh(core_axis_name="c", subcore_axis_name="s")
    out = pl.kernel(
        _emb_body,
        out_type=jax.ShapeDtypeStruct((SEQ, EMBED_DIM, n_b), jnp.float32),
        mesh=mesh,
        scratch_types=[
            pltpu.VMEM((SEQ, BLK), jnp.int32),
            pltpu.VMEM((BLK, PAD_DIM), jnp.float32),
            pltpu.VMEM((BLK, PAD_DIM), jnp.float32),
            pltpu.VMEM((EMBED_DIM, BLK), jnp.float32),
            pltpu.VMEM((EMBED_DIM, BLK), jnp.float32),
            pltpu.SemaphoreType.DMA,
            pltpu.SemaphoreType.DMA,
        ],
        compiler_params=pltpu.CompilerParams(use_tc_tiling_on_sc=True,
                                             needs_layout_passes=False),
    )(table128, idxT)
    return out.transpose(2, 0, 1)


# speedup vs baseline: 1.2493x; 1.2493x over previous
import functools

import jax
import jax.numpy as jnp
from jax import lax
from jax.experimental import pallas as pl
from jax.experimental.pallas import tpu as pltpu
from jax.experimental.pallas import tpu_sc as plsc

EMBED_DIM = 64
PAD_DIM = 128
SCALE = 8.0

_info = plsc.get_sparse_core_info()
NC, NS, L = _info.num_cores, _info.num_subcores, _info.num_lanes
NW = NC * NS

IDX_PER_STREAM = 128
STREAMS_PER_CHUNK = 2
CHUNK = IDX_PER_STREAM * STREAMS_PER_CHUNK


def _emb_body(n_chunks, table_hbm, idx_hbm, out_hbm,
              idx_all, buf0, buf1, sem0, sem1):
    wid = lax.axis_index("s") * NC + lax.axis_index("c")
    irows = n_chunks * STREAMS_PER_CHUNK
    irow0 = wid * irows
    row_base = irow0 * IDX_PER_STREAM

    pltpu.sync_copy(idx_hbm.at[pl.ds(irow0, irows)], idx_all)

    def fire(g, buf, sem):
        for j in range(STREAMS_PER_CHUNK):
            pltpu.async_copy(
                table_hbm.at[idx_all.at[g * STREAMS_PER_CHUNK + j]],
                buf.at[pl.ds(j * IDX_PER_STREAM, IDX_PER_STREAM)],
                sem,
            )

    def drain(buf, sem):
        pltpu.make_async_copy(table_hbm.at[pl.ds(0, CHUNK)], buf, sem).wait()

    def scale(buf):
        @plsc.parallel_loop(0, CHUNK, unroll=8)
        def _(r):
            for c in range(EMBED_DIM // L):
                buf[r, pl.ds(c * L, L)] = buf[r, pl.ds(c * L, L)] * SCALE

    def store(g, buf):
        pltpu.sync_copy(buf, out_hbm.at[pl.ds(row_base + g * CHUNK, CHUNK)])

    n_pairs = n_chunks // 2
    fire(0, buf0, sem0)

    def pair(p, carry):
        g0 = p * 2
        fire(g0 + 1, buf1, sem1)
        drain(buf0, sem0)
        scale(buf0)
        store(g0, buf0)
        fire(g0 + 2, buf0, sem0)
        drain(buf1, sem1)
        scale(buf1)
        store(g0 + 1, buf1)
        return carry

    lax.fori_loop(0, n_pairs - 1, pair, 0)

    g0 = (n_pairs - 1) * 2
    fire(g0 + 1, buf1, sem1)
    drain(buf0, sem0)
    scale(buf0)
    store(g0, buf0)
    drain(buf1, sem1)
    scale(buf1)
    store(g0 + 1, buf1)


def kernel(tokens, table):
    n_tok = tokens.shape[0] * tokens.shape[1]
    assert n_tok % (NW * CHUNK * 2) == 0
    n_chunks = n_tok // (NW * CHUNK)
    idx2d = tokens.reshape(n_tok // IDX_PER_STREAM, IDX_PER_STREAM)
    vocab = table.shape[0]
    table128 = jnp.concatenate(
        [table, jnp.zeros((vocab, PAD_DIM - EMBED_DIM), table.dtype)], axis=1)

    mesh = plsc.VectorSubcoreMesh(core_axis_name="c", subcore_axis_name="s")
    out = pl.kernel(
        functools.partial(_emb_body, n_chunks),
        out_type=jax.ShapeDtypeStruct((n_tok, PAD_DIM), jnp.float32),
        mesh=mesh,
        scratch_types=[
            pltpu.VMEM((n_chunks * STREAMS_PER_CHUNK, IDX_PER_STREAM),
                       jnp.int32),
            pltpu.VMEM((CHUNK, PAD_DIM), jnp.float32),
            pltpu.VMEM((CHUNK, PAD_DIM), jnp.float32),
            pltpu.SemaphoreType.DMA,
            pltpu.SemaphoreType.DMA,
        ],
        compiler_params=pltpu.CompilerParams(use_tc_tiling_on_sc=True),
    )(table128, idx2d)
    out = out[:, :EMBED_DIM]
    return out.reshape(tokens.shape[0], tokens.shape[1], EMBED_DIM)
